# Initial kernel scaffold; baseline (speedup 1.0000x reference)
import jax
import jax.numpy as jnp
from jax.experimental import pallas as pl


def kernel(xyz1, xyz2, flow1, neighr):
    raise NotImplementedError("write your pallas kernel here")


# speedup vs baseline: 37.1455x; 37.1455x over previous
import jax
import jax.numpy as jnp
from jax.experimental import pallas as pl
from jax.experimental.pallas import tpu as pltpu

_T2 = 256
_K = 3


def _warp_kernel(xyz1_ref, flow1_ref, xyz2_ref, out_ref):
    keys = xyz1_ref[0] + flow1_ref[0]
    q = xyz2_ref[0]

    d_exact = (q[0][:, None] - keys[0][None, :]) ** 2
    d_exact = d_exact + (q[1][:, None] - keys[1][None, :]) ** 2
    d_exact = d_exact + (q[2][:, None] - keys[2][None, :]) ** 2

    mm = jax.lax.dot_general(
        q.astype(jnp.bfloat16), keys.astype(jnp.bfloat16),
        dimension_numbers=(((0,), (0,)), ((), ())),
        preferred_element_type=jnp.float32,
    )
    q2 = q[0] * q[0] + q[1] * q[1] + q[2] * q[2]
    k2 = keys[0] * keys[0] + keys[1] * keys[1] + keys[2] * keys[2]
    d_sel = (-2.0 * mm + q2[:, None]) + k2[None, :]

    iota = jax.lax.broadcasted_iota(jnp.int32, d_sel.shape, 1)

    mins = []
    idxs = []
    for _ in range(_K):
        idx = jnp.argmin(d_sel, axis=1).astype(jnp.int32)
        sel = iota == idx[:, None]
        m = jnp.min(jnp.where(sel, d_exact, jnp.float32(jnp.inf)), axis=1)
        mins.append(m)
        idxs.append(idx)
        d_sel = jnp.where(sel, jnp.float32(jnp.inf), d_sel)

    dist = [jnp.maximum(jnp.sqrt(m), 1e-10) for m in mins]
    inv = [1.0 / x for x in dist]
    norm = inv[0] + inv[1] + inv[2]
    w = [x / norm for x in inv]

    wmat = jnp.where(iota == idxs[0][:, None], w[0][:, None], 0.0)
    wmat = wmat + jnp.where(iota == idxs[1][:, None], w[1][:, None], 0.0)
    wmat = wmat + jnp.where(iota == idxs[2][:, None], w[2][:, None], 0.0)

    flow2 = jax.lax.dot_general(
        flow1_ref[0], wmat,
        dimension_numbers=(((1,), (1,)), ((), ())),
        preferred_element_type=jnp.float32,
    )
    out_ref[0] = q - flow2


def kernel(xyz1, xyz2, flow1, neighr):
    del neighr
    b, _, n1 = xyz1.shape
    n2 = xyz2.shape[2]

    return pl.pallas_call(
        _warp_kernel,
        grid=(b, n2 // _T2),
        in_specs=[
            pl.BlockSpec((1, 3, n1), lambda bi, j: (bi, 0, 0)),
            pl.BlockSpec((1, 3, n1), lambda bi, j: (bi, 0, 0)),
            pl.BlockSpec((1, 3, _T2), lambda bi, j: (bi, 0, j)),
        ],
        out_specs=pl.BlockSpec((1, 3, _T2), lambda bi, j: (bi, 0, j)),
        out_shape=jax.ShapeDtypeStruct((b, 3, n2), jnp.float32),
        compiler_params=pltpu.CompilerParams(
            dimension_semantics=("parallel", "parallel"),
        ),
    )(xyz1, flow1, xyz2)


# speedup vs baseline: 51.6663x; 1.3909x over previous
import jax
import jax.numpy as jnp
from jax.experimental import pallas as pl
from jax.experimental.pallas import tpu as pltpu

_T2 = 256
_K = 3


def _warp_kernel(xyz1_ref, flow1_ref, xyz2_ref, out_ref):
    flow = flow1_ref[0]
    keys = xyz1_ref[0] + flow
    q = xyz2_ref[0]

    mm = jax.lax.dot_general(
        q.astype(jnp.bfloat16), keys.astype(jnp.bfloat16),
        dimension_numbers=(((0,), (0,)), ((), ())),
        preferred_element_type=jnp.float32,
    )
    q2 = q[0] * q[0] + q[1] * q[1] + q[2] * q[2]
    k2 = keys[0] * keys[0] + keys[1] * keys[1] + keys[2] * keys[2]
    d_sel = (-2.0 * mm + q2[:, None]) + k2[None, :]

    iota = jax.lax.broadcasted_iota(jnp.int32, d_sel.shape, 1)
    bits = jax.lax.bitcast_convert_type(d_sel, jnp.int32)
    bits = jnp.where(bits < 0, bits ^ jnp.int32(0x7FFFFFFF), bits)
    u = (bits & jnp.int32(~0xFFF)) | iota

    idxs = []
    for k in range(_K):
        umin = jnp.min(u, axis=1)
        idxs.append(umin & jnp.int32(0xFFF))
        if k < _K - 1:
            u = jnp.where(u == umin[:, None], jnp.int32(0x7FFFFFFF), u)

    kf = jnp.concatenate([keys, flow], axis=0)
    gs = []
    for k in range(_K):
        onehot = (iota == idxs[k][:, None]).astype(jnp.float32)
        gs.append(jax.lax.dot_general(
            kf, onehot,
            dimension_numbers=(((1,), (1,)), ((), ())),
            preferred_element_type=jnp.float32,
        ))

    dist = []
    for g in gs:
        dd = (g[0] - q[0]) ** 2 + (g[1] - q[1]) ** 2 + (g[2] - q[2]) ** 2
        dist.append(jnp.maximum(jnp.sqrt(dd), 1e-10))
    inv = [1.0 / x for x in dist]
    norm = inv[0] + inv[1] + inv[2]
    w = [x / norm for x in inv]

    for c in range(3):
        flow2_c = w[0] * gs[0][3 + c] + w[1] * gs[1][3 + c] + w[2] * gs[2][3 + c]
        out_ref[0, c, :] = q[c] - flow2_c


def kernel(xyz1, xyz2, flow1, neighr):
    del neighr
    b, _, n1 = xyz1.shape
    n2 = xyz2.shape[2]

    return pl.pallas_call(
        _warp_kernel,
        grid=(b, n2 // _T2),
        in_specs=[
            pl.BlockSpec((1, 3, n1), lambda bi, j: (bi, 0, 0)),
            pl.BlockSpec((1, 3, n1), lambda bi, j: (bi, 0, 0)),
            pl.BlockSpec((1, 3, _T2), lambda bi, j: (bi, 0, j)),
        ],
        out_specs=pl.BlockSpec((1, 3, _T2), lambda bi, j: (bi, 0, j)),
        out_shape=jax.ShapeDtypeStruct((b, 3, n2), jnp.float32),
        compiler_params=pltpu.CompilerParams(
            dimension_semantics=("parallel", "parallel"),
        ),
    )(xyz1, flow1, xyz2)


# speedup vs baseline: 57.9757x; 1.1221x over previous
import jax
import jax.numpy as jnp
from jax.experimental import pallas as pl
from jax.experimental.pallas import tpu as pltpu

_T2 = 256
_K = 3


def _warp_kernel(xyz1_ref, flow1_ref, xyz2_ref, out_ref):
    flow = flow1_ref[0]
    keys = xyz1_ref[0] + flow
    q = xyz2_ref[0]

    mm = jax.lax.dot_general(
        q.astype(jnp.bfloat16), keys.astype(jnp.bfloat16),
        dimension_numbers=(((0,), (0,)), ((), ())),
        preferred_element_type=jnp.float32,
    )
    q2 = q[0] * q[0] + q[1] * q[1] + q[2] * q[2]
    k2 = keys[0] * keys[0] + keys[1] * keys[1] + keys[2] * keys[2]
    d_sel = (-2.0 * mm + q2[:, None]) + k2[None, :]

    iota = jax.lax.broadcasted_iota(jnp.int32, d_sel.shape, 1)
    bits = jax.lax.bitcast_convert_type(d_sel, jnp.int32)
    u = jax.lax.bitcast_convert_type(
        (bits & jnp.int32(~0xFFF)) | iota, jnp.float32)

    inf = jnp.float32(jnp.inf)
    m1 = jnp.min(u, axis=1)
    eq1 = u == m1[:, None]
    m2 = jnp.min(jnp.where(eq1, inf, u), axis=1)
    eq2 = u == m2[:, None]
    m3 = jnp.min(jnp.where(eq1 | eq2, inf, u), axis=1)
    idxs = [jax.lax.bitcast_convert_type(m, jnp.int32) & jnp.int32(0xFFF)
            for m in (m1, m2, m3)]

    kf = jnp.concatenate([keys, flow], axis=0)
    gs = []
    for k in range(_K):
        onehot = (iota == idxs[k][:, None]).astype(jnp.float32)
        gs.append(jax.lax.dot_general(
            kf, onehot,
            dimension_numbers=(((1,), (1,)), ((), ())),
            preferred_element_type=jnp.float32,
        ))

    dist = []
    for g in gs:
        dd = (g[0] - q[0]) ** 2 + (g[1] - q[1]) ** 2 + (g[2] - q[2]) ** 2
        dist.append(jnp.maximum(jnp.sqrt(dd), 1e-10))
    inv = [1.0 / x for x in dist]
    norm = inv[0] + inv[1] + inv[2]
    w = [x / norm for x in inv]

    for c in range(3):
        flow2_c = w[0] * gs[0][3 + c] + w[1] * gs[1][3 + c] + w[2] * gs[2][3 + c]
        out_ref[0, c, :] = q[c] - flow2_c


def kernel(xyz1, xyz2, flow1, neighr):
    del neighr
    b, _, n1 = xyz1.shape
    n2 = xyz2.shape[2]

    return pl.pallas_call(
        _warp_kernel,
        grid=(b, n2 // _T2),
        in_specs=[
            pl.BlockSpec((1, 3, n1), lambda bi, j: (bi, 0, 0)),
            pl.BlockSpec((1, 3, n1), lambda bi, j: (bi, 0, 0)),
            pl.BlockSpec((1, 3, _T2), lambda bi, j: (bi, 0, j)),
        ],
        out_specs=pl.BlockSpec((1, 3, _T2), lambda bi, j: (bi, 0, j)),
        out_shape=jax.ShapeDtypeStruct((b, 3, n2), jnp.float32),
        compiler_params=pltpu.CompilerParams(
            dimension_semantics=("parallel", "parallel"),
        ),
    )(xyz1, flow1, xyz2)


# speedup vs baseline: 68.9842x; 1.1899x over previous
import jax
import jax.numpy as jnp
from jax.experimental import pallas as pl
from jax.experimental.pallas import tpu as pltpu

_T2 = 512
_K = 3


def _warp_kernel(xyz1_ref, flow1_ref, xyz2_ref, out_ref):
    flow = flow1_ref[0]
    keys = xyz1_ref[0] + flow
    q = xyz2_ref[0]

    mm = jax.lax.dot_general(
        q.astype(jnp.bfloat16), keys.astype(jnp.bfloat16),
        dimension_numbers=(((0,), (0,)), ((), ())),
        preferred_element_type=jnp.float32,
    )
    q2 = q[0] * q[0] + q[1] * q[1] + q[2] * q[2]
    k2 = keys[0] * keys[0] + keys[1] * keys[1] + keys[2] * keys[2]
    d_sel = (-2.0 * mm + q2[:, None]) + k2[None, :]

    iota = jax.lax.broadcasted_iota(jnp.int32, d_sel.shape, 1)
    bits = jax.lax.bitcast_convert_type(d_sel, jnp.int32)
    u = jax.lax.bitcast_convert_type(
        (bits & jnp.int32(~0xFFF)) | iota, jnp.float32)

    inf = jnp.float32(jnp.inf)
    m1 = jnp.min(u, axis=1)
    eq1 = u == m1[:, None]
    m2 = jnp.min(jnp.where(eq1, inf, u), axis=1)
    eq2 = u == m2[:, None]
    m3 = jnp.min(jnp.where(eq1 | eq2, inf, u), axis=1)
    idxs = [jax.lax.bitcast_convert_type(m, jnp.int32) & jnp.int32(0xFFF)
            for m in (m1, m2, m3)]

    kf = jnp.concatenate([keys, flow], axis=0)
    gs = []
    for k in range(_K):
        onehot = (iota == idxs[k][:, None]).astype(jnp.float32)
        gs.append(jax.lax.dot_general(
            kf, onehot,
            dimension_numbers=(((1,), (1,)), ((), ())),
            preferred_element_type=jnp.float32,
        ))

    dist = []
    for g in gs:
        dd = (g[0] - q[0]) ** 2 + (g[1] - q[1]) ** 2 + (g[2] - q[2]) ** 2
        dist.append(jnp.maximum(jnp.sqrt(dd), 1e-10))
    inv = [1.0 / x for x in dist]
    norm = inv[0] + inv[1] + inv[2]
    w = [x / norm for x in inv]

    for c in range(3):
        flow2_c = w[0] * gs[0][3 + c] + w[1] * gs[1][3 + c] + w[2] * gs[2][3 + c]
        out_ref[0, c, :] = q[c] - flow2_c


def kernel(xyz1, xyz2, flow1, neighr):
    del neighr
    b, _, n1 = xyz1.shape
    n2 = xyz2.shape[2]

    return pl.pallas_call(
        _warp_kernel,
        grid=(b, n2 // _T2),
        in_specs=[
            pl.BlockSpec((1, 3, n1), lambda bi, j: (bi, 0, 0)),
            pl.BlockSpec((1, 3, n1), lambda bi, j: (bi, 0, 0)),
            pl.BlockSpec((1, 3, _T2), lambda bi, j: (bi, 0, j)),
        ],
        out_specs=pl.BlockSpec((1, 3, _T2), lambda bi, j: (bi, 0, j)),
        out_shape=jax.ShapeDtypeStruct((b, 3, n2), jnp.float32),
        compiler_params=pltpu.CompilerParams(
            dimension_semantics=("parallel", "parallel"),
        ),
    )(xyz1, flow1, xyz2)


# speedup vs baseline: 69.2983x; 1.0046x over previous
import jax
import jax.numpy as jnp
from jax.experimental import pallas as pl
from jax.experimental.pallas import tpu as pltpu

_T2 = 1024
_K = 3


def _warp_kernel(xyz1_ref, flow1_ref, xyz2_ref, out_ref):
    flow = flow1_ref[0]
    keys = xyz1_ref[0] + flow
    q = xyz2_ref[0]

    mm = jax.lax.dot_general(
        q.astype(jnp.bfloat16), keys.astype(jnp.bfloat16),
        dimension_numbers=(((0,), (0,)), ((), ())),
        preferred_element_type=jnp.float32,
    )
    q2 = q[0] * q[0] + q[1] * q[1] + q[2] * q[2]
    k2 = keys[0] * keys[0] + keys[1] * keys[1] + keys[2] * keys[2]
    d_sel = (-2.0 * mm + q2[:, None]) + k2[None, :]

    iota = jax.lax.broadcasted_iota(jnp.int32, d_sel.shape, 1)
    bits = jax.lax.bitcast_convert_type(d_sel, jnp.int32)
    u = jax.lax.bitcast_convert_type(
        (bits & jnp.int32(~0xFFF)) | iota, jnp.float32)

    inf = jnp.float32(jnp.inf)
    m1 = jnp.min(u, axis=1)
    eq1 = u == m1[:, None]
    m2 = jnp.min(jnp.where(eq1, inf, u), axis=1)
    eq2 = u == m2[:, None]
    m3 = jnp.min(jnp.where(eq1 | eq2, inf, u), axis=1)
    idxs = [jax.lax.bitcast_convert_type(m, jnp.int32) & jnp.int32(0xFFF)
            for m in (m1, m2, m3)]

    kf = jnp.concatenate([keys, flow], axis=0)
    gs = []
    for k in range(_K):
        onehot = (iota == idxs[k][:, None]).astype(jnp.float32)
        gs.append(jax.lax.dot_general(
            kf, onehot,
            dimension_numbers=(((1,), (1,)), ((), ())),
            preferred_element_type=jnp.float32,
        ))

    dist = []
    for g in gs:
        dd = (g[0] - q[0]) ** 2 + (g[1] - q[1]) ** 2 + (g[2] - q[2]) ** 2
        dist.append(jnp.maximum(jnp.sqrt(dd), 1e-10))
    inv = [1.0 / x for x in dist]
    norm = inv[0] + inv[1] + inv[2]
    w = [x / norm for x in inv]

    for c in range(3):
        flow2_c = w[0] * gs[0][3 + c] + w[1] * gs[1][3 + c] + w[2] * gs[2][3 + c]
        out_ref[0, c, :] = q[c] - flow2_c


def kernel(xyz1, xyz2, flow1, neighr):
    del neighr
    b, _, n1 = xyz1.shape
    n2 = xyz2.shape[2]

    return pl.pallas_call(
        _warp_kernel,
        grid=(b, n2 // _T2),
        in_specs=[
            pl.BlockSpec((1, 3, n1), lambda bi, j: (bi, 0, 0)),
            pl.BlockSpec((1, 3, n1), lambda bi, j: (bi, 0, 0)),
            pl.BlockSpec((1, 3, _T2), lambda bi, j: (bi, 0, j)),
        ],
        out_specs=pl.BlockSpec((1, 3, _T2), lambda bi, j: (bi, 0, j)),
        out_shape=jax.ShapeDtypeStruct((b, 3, n2), jnp.float32),
        compiler_params=pltpu.CompilerParams(
            dimension_semantics=("parallel", "parallel"),
        ),
    )(xyz1, flow1, xyz2)


# speedup vs baseline: 74.4809x; 1.0748x over previous
import jax
import jax.numpy as jnp
from jax.experimental import pallas as pl
from jax.experimental.pallas import tpu as pltpu

_T2 = 1024
_K = 3


def _warp_kernel(xyz1_ref, flow1_ref, xyz2_ref, out_ref):
    flow = flow1_ref[0]
    keys = xyz1_ref[0] + flow
    q = xyz2_ref[0]

    mm = jax.lax.dot_general(
        q.astype(jnp.bfloat16), keys.astype(jnp.bfloat16),
        dimension_numbers=(((0,), (0,)), ((), ())),
        preferred_element_type=jnp.float32,
    )
    k2 = keys[0] * keys[0] + keys[1] * keys[1] + keys[2] * keys[2]
    d_sel = -2.0 * mm + k2[None, :]

    iota = jax.lax.broadcasted_iota(jnp.int32, d_sel.shape, 1)
    bits = jax.lax.bitcast_convert_type(d_sel, jnp.int32)
    u = jax.lax.bitcast_convert_type(
        (bits & jnp.int32(~0xFFF)) | iota, jnp.float32)

    inf = jnp.float32(jnp.inf)
    m1 = jnp.min(u, axis=1)
    eq1 = u == m1[:, None]
    m2 = jnp.min(jnp.where(eq1, inf, u), axis=1)
    eq2 = u == m2[:, None]
    m3 = jnp.min(jnp.where(eq1 | eq2, inf, u), axis=1)
    eq3 = u == m3[:, None]

    kf = jnp.concatenate([keys, flow], axis=0)
    gs = []
    for eq in (eq1, eq2, eq3):
        gs.append(jax.lax.dot_general(
            kf, eq.astype(jnp.float32),
            dimension_numbers=(((1,), (1,)), ((), ())),
            preferred_element_type=jnp.float32,
        ))

    dist = []
    for g in gs:
        dd = (g[0] - q[0]) ** 2 + (g[1] - q[1]) ** 2 + (g[2] - q[2]) ** 2
        dist.append(jnp.maximum(jnp.sqrt(dd), 1e-10))
    inv = [1.0 / x for x in dist]
    norm = inv[0] + inv[1] + inv[2]
    w = [x / norm for x in inv]

    for c in range(3):
        flow2_c = w[0] * gs[0][3 + c] + w[1] * gs[1][3 + c] + w[2] * gs[2][3 + c]
        out_ref[0, c, :] = q[c] - flow2_c


def kernel(xyz1, xyz2, flow1, neighr):
    del neighr
    b, _, n1 = xyz1.shape
    n2 = xyz2.shape[2]

    return pl.pallas_call(
        _warp_kernel,
        grid=(b, n2 // _T2),
        in_specs=[
            pl.BlockSpec((1, 3, n1), lambda bi, j: (bi, 0, 0)),
            pl.BlockSpec((1, 3, n1), lambda bi, j: (bi, 0, 0)),
            pl.BlockSpec((1, 3, _T2), lambda bi, j: (bi, 0, j)),
        ],
        out_specs=pl.BlockSpec((1, 3, _T2), lambda bi, j: (bi, 0, j)),
        out_shape=jax.ShapeDtypeStruct((b, 3, n2), jnp.float32),
        compiler_params=pltpu.CompilerParams(
            dimension_semantics=("parallel", "parallel"),
        ),
    )(xyz1, flow1, xyz2)
